# baseline (speedup 1.0000x reference)
import functools
import math

import jax
import jax.numpy as jnp
import numpy as np
from jax import lax
from jax.experimental import pallas as pl
from jax.experimental.pallas import tpu as pltpu
from jax.experimental.pallas import tpu_sc as plsc

B = 1024
C = 1024
N = 4096
K = 4
FT = 16
OUTW = N * FT + 1
LOG_FT = math.log(FT)

NC = 2
NS = 16
NW = NC * NS

BA = 128
BC = 32
BF = 512

_R = np.zeros((256, 1024), np.float32)
_i = np.arange(256)
for _d in range(4):
    _R[_i, (_i // 64) * 256 + (_i % 64) * 4 + _d] = 1.0
_RBLK = _R

_P = np.zeros((FT, 1024), np.float32)
_li = np.arange(1024)
_P[4 * ((_li >> 8) & 3) + (_li & 3), _li] = 1.0
_PMAT = _P


def _stage_a_body(x_ref, gin_ref, bin_ref, wn_ref, bn_ref, w1_ref, b1_ref,
                  w2_ref, b2_ref, w3_ref, b3_ref, wf1a_ref, bf1_ref,
                  coarse_ref, noop_ref, idx_ref, sel_ref, xpart_ref):
    xb = x_ref[...]
    m = jnp.mean(xb, axis=-1, keepdims=True)
    xc = xb - m
    v = jnp.mean(xc * xc, axis=-1, keepdims=True)
    xn = xc * lax.rsqrt(v + 1e-5) * gin_ref[...] + bin_ref[...]
    noop_ref[...] = jnp.dot(xn, wn_ref[...],
                            preferred_element_type=jnp.float32) + bn_ref[...]
    h = jnp.maximum(jnp.dot(xn, w1_ref[...],
                            preferred_element_type=jnp.float32) + b1_ref[...], 0.0)
    h = jnp.maximum(jnp.dot(h, w2_ref[...],
                            preferred_element_type=jnp.float32) + b2_ref[...], 0.0)
    coarse = jnp.dot(h, w3_ref[...],
                     preferred_element_type=jnp.float32) + b3_ref[...]
    coarse_ref[...] = coarse
    xpart_ref[...] = jnp.dot(xn, wf1a_ref[...],
                             preferred_element_type=jnp.float32) + bf1_ref[...]
    it = lax.broadcasted_iota(jnp.int32, coarse.shape, 1)
    vcur = coarse
    idxs = []
    sels = []
    for _ in range(K):
        mx = jnp.max(vcur, axis=-1, keepdims=True)
        am = jnp.min(jnp.where(vcur == mx, it, N), axis=-1, keepdims=True)
        idxs.append(am)
        sels.append(mx)
        vcur = jnp.where(it == am, -jnp.inf, vcur)
    idx_ref[...] = jnp.concatenate(idxs, axis=-1)
    sel_ref[...] = jnp.concatenate(sels, axis=-1)


def _stage_a(x, g_in, b_in, W_noop, b_noop, W1, b1, W2, b2, W3, b3, Wf1a, bf1):
    row = lambda i: (i, 0)
    fix = lambda i: (0, 0)
    return pl.pallas_call(
        _stage_a_body,
        grid=(B // BA,),
        in_specs=[
            pl.BlockSpec((BA, C), row),
            pl.BlockSpec((1, C), fix), pl.BlockSpec((1, C), fix),
            pl.BlockSpec((C, 1), fix), pl.BlockSpec((1, 1), fix),
            pl.BlockSpec((C, C), fix), pl.BlockSpec((1, C), fix),
            pl.BlockSpec((C, C), fix), pl.BlockSpec((1, C), fix),
            pl.BlockSpec((C, N), fix), pl.BlockSpec((1, N), fix),
            pl.BlockSpec((C, C), fix), pl.BlockSpec((1, C), fix),
        ],
        out_specs=[
            pl.BlockSpec((BA, N), row),
            pl.BlockSpec((BA, 1), row),
            pl.BlockSpec((BA, K), row),
            pl.BlockSpec((BA, K), row),
            pl.BlockSpec((BA, C), row),
        ],
        out_shape=[
            jax.ShapeDtypeStruct((B, N), jnp.float32),
            jax.ShapeDtypeStruct((B, 1), jnp.float32),
            jax.ShapeDtypeStruct((B, K), jnp.int32),
            jax.ShapeDtypeStruct((B, K), jnp.float32),
            jax.ShapeDtypeStruct((B, C), jnp.float32),
        ],
    )(x, g_in.reshape(1, C), b_in.reshape(1, C), W_noop, b_noop.reshape(1, 1),
      W1, b1.reshape(1, C), W2, b2.reshape(1, C), W3, b3.reshape(1, N),
      Wf1a, bf1.reshape(1, C))


def _stage_c_body(coarse_ref, noop_ref, idx_ref, vals_ref, rblk_ref, p_ref,
                  out_ref):
    cm = coarse_ref[...] - LOG_FT
    rb = rblk_ref[...]
    idxb = idx_ref[...]
    valsb = vals_ref[...]
    pm = p_ref[...]
    addpats = [jnp.dot(valsb[:, FT * k:FT * (k + 1)], pm,
                       preferred_element_type=jnp.float32) for k in range(K)]
    b2lane = (lax.broadcasted_iota(jnp.int32, (1, 1024), 1) >> 2) & 63
    out_ref[:, 0:1] = noop_ref[...]
    for g in range(16):
        vg = jnp.dot(cm[:, g * 256:(g + 1) * 256], rb,
                     preferred_element_type=jnp.float32)
        for al in range(4):
            v = vg[:, al * 256:(al + 1) * 256]
            chunk = jnp.concatenate([v, v, v, v], axis=-1)
            a = 4 * g + al
            nl = b2lane + a * 64
            for k in range(K):
                msk = idxb[:, k:k + 1] == nl
                chunk = jnp.where(msk, addpats[k], chunk)
            out_ref[:, 1 + 1024 * a: 1 + 1024 * (a + 1)] = chunk


def _stage_c(coarse, noop, idx, vals_r):
    row = lambda i: (i, 0)
    fix = lambda i: (0, 0)
    return pl.pallas_call(
        _stage_c_body,
        grid=(B // BC,),
        in_specs=[
            pl.BlockSpec((BC, N), row),
            pl.BlockSpec((BC, 1), row),
            pl.BlockSpec((BC, K), row),
            pl.BlockSpec((BC, K * FT), row),
            pl.BlockSpec((256, 1024), fix),
            pl.BlockSpec((FT, 1024), fix),
        ],
        out_specs=pl.BlockSpec((BC, OUTW), row),
        out_shape=jax.ShapeDtypeStruct((B, OUTW), jnp.float32),
    )(coarse, noop, idx, vals_r, _RBLK, _PMAT)


def _stage_f_body(e_ref, xp_ref, sel_ref, ge_ref, be_ref,
                  wf1b_ref, wf2_ref, bf2_ref, wf3_ref, bf3_ref,
                  vals_ref):
    e = e_ref[...]
    m = jnp.mean(e, axis=-1, keepdims=True)
    ec = e - m
    v = jnp.mean(ec * ec, axis=-1, keepdims=True)
    en = ec * lax.rsqrt(v + 1e-5) * ge_ref[...] + be_ref[...]
    xp = xp_ref[...]
    xp4 = jnp.broadcast_to(xp[:, None, :], (BF // K, K, C)).reshape(BF, C)
    h = jnp.maximum(jnp.dot(en, wf1b_ref[...],
                            preferred_element_type=jnp.float32) + xp4, 0.0)
    h = jnp.maximum(jnp.dot(h, wf2_ref[...],
                            preferred_element_type=jnp.float32) + bf2_ref[...], 0.0)
    fine = jnp.dot(h, wf3_ref[...],
                   preferred_element_type=jnp.float32) + bf3_ref[...]
    mx = jnp.max(fine, axis=-1, keepdims=True)
    lse = mx + jnp.log(jnp.sum(jnp.exp(fine - mx), axis=-1, keepdims=True))
    vals_ref[...] = sel_ref[...] + fine - lse


def _stage_f(e, xpart, sel_flat, g_e, b_e, Wf1b, Wf2, bf2, Wf3, bf3):
    row = lambda i: (i, 0)
    fix = lambda i: (0, 0)
    return pl.pallas_call(
        _stage_f_body,
        grid=(B * K // BF,),
        in_specs=[
            pl.BlockSpec((BF, C), row),
            pl.BlockSpec((BF // K, C), row),
            pl.BlockSpec((BF, 1), row),
            pl.BlockSpec((1, C), fix), pl.BlockSpec((1, C), fix),
            pl.BlockSpec((C, C), fix),
            pl.BlockSpec((C, C), fix), pl.BlockSpec((1, C), fix),
            pl.BlockSpec((C, FT), fix), pl.BlockSpec((1, FT), fix),
        ],
        out_specs=pl.BlockSpec((BF, FT), row),
        out_shape=jax.ShapeDtypeStruct((B * K, FT), jnp.float32),
    )(e, xpart, sel_flat, g_e.reshape(1, C), b_e.reshape(1, C),
      Wf1b, Wf2, bf2.reshape(1, C), Wf3, bf3.reshape(1, FT))


_ROWS_PER_W = B * K // NW
_GCHUNK = 64


@functools.cache
def _sc_kernels():
    mesh = plsc.VectorSubcoreMesh(core_axis_name="c", subcore_axis_name="s",
                                  num_cores=NC, num_subcores=NS)

    @functools.partial(
        pl.kernel,
        out_type=jax.ShapeDtypeStruct((B * K, C), jnp.float32),
        mesh=mesh,
        scratch_types=[
            pltpu.VMEM((_ROWS_PER_W // _GCHUNK, _GCHUNK), jnp.int32),
            pltpu.VMEM((_GCHUNK, C), jnp.float32),
            pltpu.SemaphoreType.DMA,
        ],
    )
    def sc_gather(emb_hbm, idx_hbm, e_hbm, idxv, rowsv, sem):
        wid = lax.axis_index("s") * NC + lax.axis_index("c")
        pltpu.sync_copy(idx_hbm.at[wid], idxv)
        for half in range(_ROWS_PER_W // _GCHUNK):
            pltpu.async_copy(emb_hbm.at[idxv.at[half]], rowsv, sem).wait()
            pltpu.sync_copy(rowsv,
                            e_hbm.at[pl.ds(wid * _ROWS_PER_W + half * _GCHUNK,
                                           _GCHUNK)])

    return sc_gather


def _sc_gather(emb, idx3):
    return _sc_kernels()(emb, idx3)


def kernel(x, g_in, b_in, W_noop, b_noop, W1, b1, W2, b2, W3, b3, emb,
           g_e, b_e, Wf1, bf1, Wf2, bf2, Wf3, bf3):
    Wf1a = Wf1[:C]
    Wf1b = Wf1[C:]
    coarse, noop, idx, sel, xpart = _stage_a(
        x, g_in, b_in, W_noop, b_noop, W1, b1, W2, b2, W3, b3, Wf1a, bf1)
    e = _sc_gather(emb, idx.reshape(NW, _ROWS_PER_W // _GCHUNK, _GCHUNK))
    vals = _stage_f(e, xpart, sel.reshape(B * K, 1), g_e, b_e, Wf1b, Wf2,
                    bf2, Wf3, bf3)
    return _stage_c(coarse, noop, idx, vals.reshape(B, K * FT))


# speedup vs baseline: 1.0236x; 1.0236x over previous
import functools
import math

import jax
import jax.numpy as jnp
import numpy as np
from jax import lax
from jax.experimental import pallas as pl
from jax.experimental.pallas import tpu as pltpu
from jax.experimental.pallas import tpu_sc as plsc

B = 1024
C = 1024
N = 4096
K = 4
FT = 16
OUTW = N * FT + 1
LOG_FT = math.log(FT)

NC = 2
NS = 16
NW = NC * NS

BA = 256
BC = 64
BF = 512

_R = np.zeros((256, 1024), np.float32)
_i = np.arange(256)
for _d in range(4):
    _R[_i, (_i // 64) * 256 + (_i % 64) * 4 + _d] = 1.0
_RBLK = _R

_P = np.zeros((FT, 1024), np.float32)
_li = np.arange(1024)
_P[4 * ((_li >> 8) & 3) + (_li & 3), _li] = 1.0
_PMAT = _P


def _stage_a_body(x_ref, gin_ref, bin_ref, wn_ref, bn_ref, w1_ref, b1_ref,
                  w2_ref, b2_ref, w3_ref, b3_ref, wf1a_ref, bf1_ref,
                  coarse_ref, noop_ref, idx_ref, sel_ref, xpart_ref):
    xb = x_ref[...]
    m = jnp.mean(xb, axis=-1, keepdims=True)
    xc = xb - m
    v = jnp.mean(xc * xc, axis=-1, keepdims=True)
    xn = xc * lax.rsqrt(v + 1e-5) * gin_ref[...] + bin_ref[...]
    noop_ref[...] = jnp.dot(xn, wn_ref[...],
                            preferred_element_type=jnp.float32) + bn_ref[...]
    h = jnp.maximum(jnp.dot(xn, w1_ref[...],
                            preferred_element_type=jnp.float32) + b1_ref[...], 0.0)
    h = jnp.maximum(jnp.dot(h, w2_ref[...],
                            preferred_element_type=jnp.float32) + b2_ref[...], 0.0)
    coarse = jnp.dot(h, w3_ref[...],
                     preferred_element_type=jnp.float32) + b3_ref[...]
    coarse_ref[...] = coarse
    xpart_ref[...] = jnp.dot(xn, wf1a_ref[...],
                             preferred_element_type=jnp.float32) + bf1_ref[...]
    it = lax.broadcasted_iota(jnp.int32, coarse.shape, 1)
    vcur = coarse
    idxs = []
    sels = []
    for _ in range(K):
        mx = jnp.max(vcur, axis=-1, keepdims=True)
        am = jnp.min(jnp.where(vcur == mx, it, N), axis=-1, keepdims=True)
        idxs.append(am)
        sels.append(mx)
        vcur = jnp.where(it == am, -jnp.inf, vcur)
    idx_ref[...] = jnp.concatenate(idxs, axis=-1)
    sel_ref[...] = jnp.concatenate(sels, axis=-1)


def _stage_a(x, g_in, b_in, W_noop, b_noop, W1, b1, W2, b2, W3, b3, Wf1a, bf1):
    row = lambda i: (i, 0)
    fix = lambda i: (0, 0)
    return pl.pallas_call(
        _stage_a_body,
        grid=(B // BA,),
        in_specs=[
            pl.BlockSpec((BA, C), row),
            pl.BlockSpec((1, C), fix), pl.BlockSpec((1, C), fix),
            pl.BlockSpec((C, 1), fix), pl.BlockSpec((1, 1), fix),
            pl.BlockSpec((C, C), fix), pl.BlockSpec((1, C), fix),
            pl.BlockSpec((C, C), fix), pl.BlockSpec((1, C), fix),
            pl.BlockSpec((C, N), fix), pl.BlockSpec((1, N), fix),
            pl.BlockSpec((C, C), fix), pl.BlockSpec((1, C), fix),
        ],
        out_specs=[
            pl.BlockSpec((BA, N), row),
            pl.BlockSpec((BA, 1), row),
            pl.BlockSpec((BA, K), row),
            pl.BlockSpec((BA, K), row),
            pl.BlockSpec((BA, C), row),
        ],
        out_shape=[
            jax.ShapeDtypeStruct((B, N), jnp.float32),
            jax.ShapeDtypeStruct((B, 1), jnp.float32),
            jax.ShapeDtypeStruct((B, K), jnp.int32),
            jax.ShapeDtypeStruct((B, K), jnp.float32),
            jax.ShapeDtypeStruct((B, C), jnp.float32),
        ],
    )(x, g_in.reshape(1, C), b_in.reshape(1, C), W_noop, b_noop.reshape(1, 1),
      W1, b1.reshape(1, C), W2, b2.reshape(1, C), W3, b3.reshape(1, N),
      Wf1a, bf1.reshape(1, C))


def _stage_c_body(coarse_ref, noop_ref, idx_ref, vals_ref, rblk_ref, p_ref,
                  out_ref):
    cm = coarse_ref[...] - LOG_FT
    rb = rblk_ref[...]
    idxb = idx_ref[...]
    valsb = vals_ref[...]
    pm = p_ref[...]
    addpats = [jnp.dot(valsb[:, FT * k:FT * (k + 1)], pm,
                       preferred_element_type=jnp.float32) for k in range(K)]
    b2lane = (lax.broadcasted_iota(jnp.int32, (1, 1024), 1) >> 2) & 63
    out_ref[:, 0:1] = noop_ref[...]
    for g in range(16):
        vg = jnp.dot(cm[:, g * 256:(g + 1) * 256], rb,
                     preferred_element_type=jnp.float32)
        for al in range(4):
            v = vg[:, al * 256:(al + 1) * 256]
            chunk = jnp.concatenate([v, v, v, v], axis=-1)
            a = 4 * g + al
            nl = b2lane + a * 64
            for k in range(K):
                msk = idxb[:, k:k + 1] == nl
                chunk = jnp.where(msk, addpats[k], chunk)
            out_ref[:, 1 + 1024 * a: 1 + 1024 * (a + 1)] = chunk


def _stage_c(coarse, noop, idx, vals_r):
    row = lambda i: (i, 0)
    fix = lambda i: (0, 0)
    return pl.pallas_call(
        _stage_c_body,
        grid=(B // BC,),
        in_specs=[
            pl.BlockSpec((BC, N), row),
            pl.BlockSpec((BC, 1), row),
            pl.BlockSpec((BC, K), row),
            pl.BlockSpec((BC, K * FT), row),
            pl.BlockSpec((256, 1024), fix),
            pl.BlockSpec((FT, 1024), fix),
        ],
        out_specs=pl.BlockSpec((BC, OUTW), row),
        out_shape=jax.ShapeDtypeStruct((B, OUTW), jnp.float32),
    )(coarse, noop, idx, vals_r, _RBLK, _PMAT)


def _stage_f_body(e_ref, xp_ref, sel_ref, ge_ref, be_ref,
                  wf1b_ref, wf2_ref, bf2_ref, wf3_ref, bf3_ref,
                  vals_ref):
    e = e_ref[...]
    m = jnp.mean(e, axis=-1, keepdims=True)
    ec = e - m
    v = jnp.mean(ec * ec, axis=-1, keepdims=True)
    en = ec * lax.rsqrt(v + 1e-5) * ge_ref[...] + be_ref[...]
    xp = xp_ref[...]
    xp4 = jnp.broadcast_to(xp[:, None, :], (BF // K, K, C)).reshape(BF, C)
    h = jnp.maximum(jnp.dot(en, wf1b_ref[...],
                            preferred_element_type=jnp.float32) + xp4, 0.0)
    h = jnp.maximum(jnp.dot(h, wf2_ref[...],
                            preferred_element_type=jnp.float32) + bf2_ref[...], 0.0)
    fine = jnp.dot(h, wf3_ref[...],
                   preferred_element_type=jnp.float32) + bf3_ref[...]
    mx = jnp.max(fine, axis=-1, keepdims=True)
    lse = mx + jnp.log(jnp.sum(jnp.exp(fine - mx), axis=-1, keepdims=True))
    vals_ref[...] = sel_ref[...] + fine - lse


def _stage_f(e, xpart, sel_flat, g_e, b_e, Wf1b, Wf2, bf2, Wf3, bf3):
    row = lambda i: (i, 0)
    fix = lambda i: (0, 0)
    return pl.pallas_call(
        _stage_f_body,
        grid=(B * K // BF,),
        in_specs=[
            pl.BlockSpec((BF, C), row),
            pl.BlockSpec((BF // K, C), row),
            pl.BlockSpec((BF, 1), row),
            pl.BlockSpec((1, C), fix), pl.BlockSpec((1, C), fix),
            pl.BlockSpec((C, C), fix),
            pl.BlockSpec((C, C), fix), pl.BlockSpec((1, C), fix),
            pl.BlockSpec((C, FT), fix), pl.BlockSpec((1, FT), fix),
        ],
        out_specs=pl.BlockSpec((BF, FT), row),
        out_shape=jax.ShapeDtypeStruct((B * K, FT), jnp.float32),
    )(e, xpart, sel_flat, g_e.reshape(1, C), b_e.reshape(1, C),
      Wf1b, Wf2, bf2.reshape(1, C), Wf3, bf3.reshape(1, FT))


_ROWS_PER_W = B * K // NW
_GCHUNK = 32
_NCHUNK = _ROWS_PER_W // _GCHUNK


@functools.cache
def _sc_kernels():
    mesh = plsc.VectorSubcoreMesh(core_axis_name="c", subcore_axis_name="s",
                                  num_cores=NC, num_subcores=NS)

    @functools.partial(
        pl.kernel,
        out_type=jax.ShapeDtypeStruct((B * K, C), jnp.float32),
        mesh=mesh,
        scratch_types=[
            pltpu.VMEM((_NCHUNK, _GCHUNK), jnp.int32),
            pltpu.VMEM((_GCHUNK, C), jnp.float32),
            pltpu.VMEM((_GCHUNK, C), jnp.float32),
            pltpu.SemaphoreType.DMA,
            pltpu.SemaphoreType.DMA,
        ],
    )
    def sc_gather(emb_hbm, idx_hbm, e_hbm, idxv, buf0, buf1, gsem, wsem):
        wid = lax.axis_index("s") * NC + lax.axis_index("c")
        pltpu.sync_copy(idx_hbm.at[wid], idxv)
        bufs = (buf0, buf1)
        writes = [None] * _NCHUNK
        for c in range(_NCHUNK):
            buf = bufs[c % 2]
            if c >= 2:
                writes[c - 2].wait()
            pltpu.async_copy(emb_hbm.at[idxv.at[c]], buf, gsem).wait()
            writes[c] = pltpu.async_copy(
                buf, e_hbm.at[pl.ds(wid * _ROWS_PER_W + c * _GCHUNK,
                                    _GCHUNK)], wsem)
        writes[_NCHUNK - 2].wait()
        writes[_NCHUNK - 1].wait()

    return sc_gather


def _sc_gather(emb, idx3):
    return _sc_kernels()(emb, idx3)


def kernel(x, g_in, b_in, W_noop, b_noop, W1, b1, W2, b2, W3, b3, emb,
           g_e, b_e, Wf1, bf1, Wf2, bf2, Wf3, bf3):
    Wf1a = Wf1[:C]
    Wf1b = Wf1[C:]
    coarse, noop, idx, sel, xpart = _stage_a(
        x, g_in, b_in, W_noop, b_noop, W1, b1, W2, b2, W3, b3, Wf1a, bf1)
    e = _sc_gather(emb, idx.reshape(NW, _NCHUNK, _GCHUNK))
    vals = _stage_f(e, xpart, sel.reshape(B * K, 1), g_e, b_e, Wf1b, Wf2,
                    bf2, Wf3, bf3)
    return _stage_c(coarse, noop, idx, vals.reshape(B, K * FT))


# speedup vs baseline: 1.8571x; 1.8143x over previous
import functools
import math

import jax
import jax.numpy as jnp
from jax import lax
from jax.experimental import pallas as pl
from jax.experimental.pallas import tpu as pltpu
from jax.experimental.pallas import tpu_sc as plsc

B = 1024
C = 1024
N = 4096
K = 4
FT = 16
OUTW = N * FT + 1
LOG_FT = math.log(FT)

NC = 2
NS = 16
NW = NC * NS

BA = 256
BF = 512

BCT = 2048
_NJ = -(-OUTW // BCT)
_APB = BCT // 1024


def _stage_a_body(x_ref, gin_ref, bin_ref, wn_ref, bn_ref, w1_ref, b1_ref,
                  w2_ref, b2_ref, w3_ref, b3_ref, wf1a_ref, bf1_ref,
                  coarse_t_ref, noop_ref, idx_ref, sel_ref, xpart_ref):
    xb = x_ref[...]
    m = jnp.mean(xb, axis=-1, keepdims=True)
    xc = xb - m
    v = jnp.mean(xc * xc, axis=-1, keepdims=True)
    xn = xc * lax.rsqrt(v + 1e-5) * gin_ref[...] + bin_ref[...]
    noop_ref[...] = jnp.dot(xn, wn_ref[...],
                            preferred_element_type=jnp.float32) + bn_ref[...]
    h = jnp.maximum(jnp.dot(xn, w1_ref[...],
                            preferred_element_type=jnp.float32) + b1_ref[...], 0.0)
    h = jnp.maximum(jnp.dot(h, w2_ref[...],
                            preferred_element_type=jnp.float32) + b2_ref[...], 0.0)
    coarse = jnp.dot(h, w3_ref[...],
                     preferred_element_type=jnp.float32) + b3_ref[...]
    coarse_t_ref[...] = coarse.T
    xpart_ref[...] = jnp.dot(xn, wf1a_ref[...],
                             preferred_element_type=jnp.float32) + bf1_ref[...]
    it = lax.broadcasted_iota(jnp.int32, coarse.shape, 1)
    vcur = coarse
    idxs = []
    sels = []
    for _ in range(K):
        mx = jnp.max(vcur, axis=-1, keepdims=True)
        am = jnp.min(jnp.where(vcur == mx, it, N), axis=-1, keepdims=True)
        idxs.append(am)
        sels.append(mx)
        vcur = jnp.where(it == am, -jnp.inf, vcur)
    idx_ref[...] = jnp.concatenate(idxs, axis=-1)
    sel_ref[...] = jnp.concatenate(sels, axis=-1)


def _stage_a(x, g_in, b_in, W_noop, b_noop, W1, b1, W2, b2, W3, b3, Wf1a, bf1):
    row = lambda i: (i, 0)
    fix = lambda i: (0, 0)
    return pl.pallas_call(
        _stage_a_body,
        grid=(B // BA,),
        in_specs=[
            pl.BlockSpec((BA, C), row),
            pl.BlockSpec((1, C), fix), pl.BlockSpec((1, C), fix),
            pl.BlockSpec((C, 1), fix), pl.BlockSpec((1, 1), fix),
            pl.BlockSpec((C, C), fix), pl.BlockSpec((1, C), fix),
            pl.BlockSpec((C, C), fix), pl.BlockSpec((1, C), fix),
            pl.BlockSpec((C, N), fix), pl.BlockSpec((1, N), fix),
            pl.BlockSpec((C, C), fix), pl.BlockSpec((1, C), fix),
        ],
        out_specs=[
            pl.BlockSpec((N, BA), lambda i: (0, i)),
            pl.BlockSpec((BA, 1), row),
            pl.BlockSpec((BA, K), row),
            pl.BlockSpec((BA, K), row),
            pl.BlockSpec((BA, C), row),
        ],
        out_shape=[
            jax.ShapeDtypeStruct((N, B), jnp.float32),
            jax.ShapeDtypeStruct((B, 1), jnp.float32),
            jax.ShapeDtypeStruct((B, K), jnp.int32),
            jax.ShapeDtypeStruct((B, K), jnp.float32),
            jax.ShapeDtypeStruct((B, C), jnp.float32),
        ],
    )(x, g_in.reshape(1, C), b_in.reshape(1, C), W_noop, b_noop.reshape(1, 1),
      W1, b1.reshape(1, C), W2, b2.reshape(1, C), W3, b3.reshape(1, N),
      Wf1a, bf1.reshape(1, C))


def _stage_c_body(cur_ref, prev_ref, noop_ref, idx_ref, vals_ref,
                  out_ref, *pats):
    j = pl.program_id(0)
    t = lax.broadcasted_iota(jnp.int32, (1024, 1), 0)
    b2v = (t >> 2) & 63

    @pl.when(j == 0)
    def _fill_pats():
        for k in range(K):
            cparts = []
            for c in range(4):
                grp = vals_ref[FT * k + 4 * c: FT * k + 4 * c + 4, :]
                cparts.append(jnp.broadcast_to(grp[None, :, :],
                                               (64, 4, B)).reshape(256, B))
            pats[k][...] = jnp.concatenate(cparts, axis=0)

    ct = cur_ref[...]
    segs = []
    for a_loc in range(_APB):
        ca = ct[64 * a_loc: 64 * (a_loc + 1), :]
        rep = jnp.broadcast_to(ca[:, None, :], (64, 4, B)).reshape(256, B)
        seg = jnp.concatenate([rep, rep, rep, rep], axis=0) - LOG_FT
        n_vec = (_APB * j + a_loc) * 64 + b2v
        for k in range(K):
            msk = idx_ref[k:k + 1, :] == n_vec
            seg = jnp.where(msk, pats[k][...], seg)
        segs.append(seg)
    X = jnp.concatenate(segs, axis=0)
    npv = BCT // FT * j - 1
    row0 = prev_ref[127:128, :] - LOG_FT
    for k in range(K):
        m = idx_ref[k:k + 1, :] == npv
        row0 = jnp.where(m, vals_ref[FT * k + FT - 1: FT * k + FT, :], row0)
    row0 = jnp.where(j == 0, noop_ref[...], row0)
    out_ref[...] = jnp.concatenate([row0, X[:BCT - 1, :]], axis=0)


def _stage_c(coarse_t, noop_t, idx_t, vals_t):
    fix = lambda j: (0, 0)
    nct = BCT // FT
    return pl.pallas_call(
        _stage_c_body,
        grid=(_NJ,),
        in_specs=[
            pl.BlockSpec((nct, B), lambda j: (jnp.minimum(j, N // nct - 1), 0)),
            pl.BlockSpec((nct, B), lambda j: (jnp.maximum(j - 1, 0), 0)),
            pl.BlockSpec((1, B), fix),
            pl.BlockSpec((K, B), fix),
            pl.BlockSpec((K * FT, B), fix),
        ],
        out_specs=pl.BlockSpec((BCT, B), lambda j: (j, 0)),
        out_shape=jax.ShapeDtypeStruct((OUTW, B), jnp.float32),
        scratch_shapes=[pltpu.VMEM((1024, B), jnp.float32)
                        for _ in range(K)],
    )(coarse_t, coarse_t, noop_t, idx_t, vals_t)


def _stage_f_body(e_ref, xp_ref, sel_ref, ge_ref, be_ref,
                  wf1b_ref, wf2_ref, bf2_ref, wf3_ref, bf3_ref,
                  vals_ref):
    e = e_ref[...]
    m = jnp.mean(e, axis=-1, keepdims=True)
    ec = e - m
    v = jnp.mean(ec * ec, axis=-1, keepdims=True)
    en = ec * lax.rsqrt(v + 1e-5) * ge_ref[...] + be_ref[...]
    xp = xp_ref[...]
    xp4 = jnp.broadcast_to(xp[:, None, :], (BF // K, K, C)).reshape(BF, C)
    h = jnp.maximum(jnp.dot(en, wf1b_ref[...],
                            preferred_element_type=jnp.float32) + xp4, 0.0)
    h = jnp.maximum(jnp.dot(h, wf2_ref[...],
                            preferred_element_type=jnp.float32) + bf2_ref[...], 0.0)
    fine = jnp.dot(h, wf3_ref[...],
                   preferred_element_type=jnp.float32) + bf3_ref[...]
    mx = jnp.max(fine, axis=-1, keepdims=True)
    lse = mx + jnp.log(jnp.sum(jnp.exp(fine - mx), axis=-1, keepdims=True))
    vals_ref[...] = sel_ref[...] + fine - lse


def _stage_f(e, xpart, sel_flat, g_e, b_e, Wf1b, Wf2, bf2, Wf3, bf3):
    row = lambda i: (i, 0)
    fix = lambda i: (0, 0)
    return pl.pallas_call(
        _stage_f_body,
        grid=(B * K // BF,),
        in_specs=[
            pl.BlockSpec((BF, C), row),
            pl.BlockSpec((BF // K, C), row),
            pl.BlockSpec((BF, 1), row),
            pl.BlockSpec((1, C), fix), pl.BlockSpec((1, C), fix),
            pl.BlockSpec((C, C), fix),
            pl.BlockSpec((C, C), fix), pl.BlockSpec((1, C), fix),
            pl.BlockSpec((C, FT), fix), pl.BlockSpec((1, FT), fix),
        ],
        out_specs=pl.BlockSpec((BF, FT), row),
        out_shape=jax.ShapeDtypeStruct((B * K, FT), jnp.float32),
    )(e, xpart, sel_flat, g_e.reshape(1, C), b_e.reshape(1, C),
      Wf1b, Wf2, bf2.reshape(1, C), Wf3, bf3.reshape(1, FT))


_ROWS_PER_W = B * K // NW
_GCHUNK = 32
_NCHUNK = _ROWS_PER_W // _GCHUNK


@functools.cache
def _sc_kernels():
    mesh = plsc.VectorSubcoreMesh(core_axis_name="c", subcore_axis_name="s",
                                  num_cores=NC, num_subcores=NS)

    @functools.partial(
        pl.kernel,
        out_type=jax.ShapeDtypeStruct((B * K, C), jnp.float32),
        mesh=mesh,
        scratch_types=[
            pltpu.VMEM((_NCHUNK, _GCHUNK), jnp.int32),
            pltpu.VMEM((_GCHUNK, C), jnp.float32),
            pltpu.VMEM((_GCHUNK, C), jnp.float32),
            pltpu.SemaphoreType.DMA,
            pltpu.SemaphoreType.DMA,
        ],
    )
    def sc_gather(emb_hbm, idx_hbm, e_hbm, idxv, buf0, buf1, gsem, wsem):
        wid = lax.axis_index("s") * NC + lax.axis_index("c")
        pltpu.sync_copy(idx_hbm.at[wid], idxv)
        bufs = (buf0, buf1)
        writes = [None] * _NCHUNK
        for c in range(_NCHUNK):
            buf = bufs[c % 2]
            if c >= 2:
                writes[c - 2].wait()
            pltpu.async_copy(emb_hbm.at[idxv.at[c]], buf, gsem).wait()
            writes[c] = pltpu.async_copy(
                buf, e_hbm.at[pl.ds(wid * _ROWS_PER_W + c * _GCHUNK,
                                    _GCHUNK)], wsem)
        writes[_NCHUNK - 2].wait()
        writes[_NCHUNK - 1].wait()

    return sc_gather


def _sc_gather(emb, idx3):
    return _sc_kernels()(emb, idx3)


def kernel(x, g_in, b_in, W_noop, b_noop, W1, b1, W2, b2, W3, b3, emb,
           g_e, b_e, Wf1, bf1, Wf2, bf2, Wf3, bf3):
    Wf1a = Wf1[:C]
    Wf1b = Wf1[C:]
    coarse_t, noop, idx, sel, xpart = _stage_a(
        x, g_in, b_in, W_noop, b_noop, W1, b1, W2, b2, W3, b3, Wf1a, bf1)
    e = _sc_gather(emb, idx.reshape(NW, _NCHUNK, _GCHUNK))
    vals = _stage_f(e, xpart, sel.reshape(B * K, 1), g_e, b_e, Wf1b, Wf2,
                    bf2, Wf3, bf3)
    out_t = _stage_c(coarse_t, noop.reshape(1, B), idx.T,
                     vals.reshape(B, K * FT).T)
    return out_t.T
